# baseline (speedup 1.0000x reference)
import functools

import jax
import jax.numpy as jnp
from jax import lax
from jax.experimental import pallas as pl
from jax.experimental.pallas import tpu as pltpu
from jax.experimental.pallas import tpu_sc as plsc

NW = 32
NC = 2
CHUNK = 128
LANES = 16
NBUF = 2


@functools.partial(jax.jit, static_argnames=("C", "D", "H", "W"))
def _run(pts_flat, table, coef, *, C, D, H, W):
    P = pts_flat.shape[0] // 3
    T = -(-P // CHUNK)
    CPW = -(-(-(-T // NW)) // NBUF) * NBUF
    HW = H * W

    mesh = plsc.VectorSubcoreMesh(core_axis_name="c", subcore_axis_name="s")

    @functools.partial(
        pl.kernel,
        mesh=mesh,
        compiler_params=pltpu.CompilerParams(
            needs_layout_passes=False, use_tc_tiling_on_sc=False),
        out_type=jax.ShapeDtypeStruct((P * C // 128, 128), jnp.float32),
        scratch_types=[
            pltpu.VMEM((8, LANES), jnp.float32),
            [pltpu.VMEM((3 * CHUNK,), jnp.float32) for _ in range(NBUF)],
            [[pltpu.VMEM((CHUNK,), jnp.int32) for _ in range(8)]
             for _ in range(NBUF)],
            [[pltpu.VMEM((CHUNK,), jnp.float32) for _ in range(8)]
             for _ in range(NBUF)],
            [[pltpu.VMEM((CHUNK, C), jnp.float32) for _ in range(8)]
             for _ in range(NBUF)],
            [pltpu.VMEM((CHUNK * C // 128, 128), jnp.float32)
             for _ in range(NBUF)],
            [pltpu.SemaphoreType.DMA for _ in range(NBUF)],
            [pltpu.SemaphoreType.DMA for _ in range(NBUF)],
            [pltpu.SemaphoreType.DMA for _ in range(NBUF)],
        ],
    )
    def grid_kernel(pts_hbm, table_hbm, coef_hbm, out_hbm,
                    coef_v, pts_v, idx_v, w_v, rows_v, out_v,
                    gsem, psem, osem):
        wid = lax.axis_index("s") * NC + lax.axis_index("c")
        pltpu.sync_copy(coef_hbm, coef_v)
        sxv = coef_v[0, :]
        syv = coef_v[1, :]
        szv = coef_v[2, :]
        oxv = coef_v[3, :]
        oyv = coef_v[4, :]
        ozv = coef_v[5, :]
        iota = lax.iota(jnp.int32, LANES)
        wbase = wid * CPW

        def chunk_base(ci):
            t = jnp.minimum(wbase + ci, T - 1)
            return jnp.minimum(t * CHUNK, P - CHUNK)

        def fire_pts(ci, b):
            base = chunk_base(ci)
            pltpu.async_copy(pts_hbm.at[pl.ds(base * 3, CHUNK * 3)],
                             pts_v[b], psem[b])

        def wait_pts(ci, b):
            base = chunk_base(ci)
            pltpu.make_async_copy(pts_hbm.at[pl.ds(base * 3, CHUNK * 3)],
                                  pts_v[b], psem[b]).wait()

        def fire_out(ci, b):
            base = chunk_base(ci)
            pltpu.async_copy(out_v[b],
                             out_hbm.at[pl.ds(base * C // 128, CHUNK * C // 128)],
                             osem[b])

        def wait_out(ci, b):
            base = chunk_base(ci)
            pltpu.make_async_copy(
                out_v[b],
                out_hbm.at[pl.ds(base * C // 128, CHUNK * C // 128)],
                osem[b]).wait()

        def prepare(ci, b):
            wait_pts(ci, b)

            def grp_body(g):
                s = g * LANES
                p3 = (s + iota) * 3
                px = plsc.load_gather(pts_v[b], [p3])
                py = plsc.load_gather(pts_v[b], [p3 + 1])
                pz = plsc.load_gather(pts_v[b], [p3 + 2])
                ix = jnp.clip(px * sxv + oxv, 0.0, float(W - 1))
                iy = jnp.clip(py * syv + oyv, 0.0, float(H - 1))
                iz = jnp.clip(pz * szv + ozv, 0.0, float(D - 1))
                x0 = ix.astype(jnp.int32)
                y0 = iy.astype(jnp.int32)
                z0 = iz.astype(jnp.int32)
                fx = ix - x0.astype(jnp.float32)
                fy = iy - y0.astype(jnp.float32)
                fz = iz - z0.astype(jnp.float32)
                x1 = jnp.minimum(x0 + 1, W - 1)
                y1 = jnp.minimum(y0 + 1, H - 1)
                z1 = jnp.minimum(z0 + 1, D - 1)
                b00 = z0 * HW + y0 * W
                b01 = z0 * HW + y1 * W
                b10 = z1 * HW + y0 * W
                b11 = z1 * HW + y1 * W
                gx = 1.0 - fx
                a = (1.0 - fz) * (1.0 - fy)
                bb = (1.0 - fz) * fy
                c = fz * (1.0 - fy)
                d = fz * fy
                ids = (b00 + x0, b00 + x1, b01 + x0, b01 + x1,
                       b10 + x0, b10 + x1, b11 + x0, b11 + x1)
                ws = (a * gx, a * fx, bb * gx, bb * fx,
                      c * gx, c * fx, d * gx, d * fx)
                for k in range(8):
                    idx_v[b][k][pl.ds(s, LANES)] = ids[k]
                    w_v[b][k][pl.ds(s, LANES)] = ws[k]

            plsc.parallel_loop(0, CHUNK // LANES, 1)(grp_body)
            for k in range(8):
                pltpu.async_copy(table_hbm.at[idx_v[b][k]], rows_v[b][k],
                                 gsem[b])

        def consume(ci, b):
            for k in range(8):
                pltpu.make_async_copy(table_hbm.at[idx_v[b][k]],
                                      rows_v[b][k], gsem[b]).wait()

            @pl.when(ci >= NBUF)
            def _():
                wait_out(ci - NBUF, b)

            def comb_body(g):
                s = g * LANES
                ridx = s + iota
                wv = [w_v[b][k][pl.ds(s, LANES)] for k in range(8)]
                for d in range(C):
                    cv = (iota + d) & (C - 1)
                    ld = [plsc.load_gather(rows_v[b][k], [ridx, cv])
                          for k in range(8)]
                    pr = [wv[2 * j] * ld[2 * j] + wv[2 * j + 1] * ld[2 * j + 1]
                          for j in range(4)]
                    acc = (pr[0] + pr[1]) + (pr[2] + pr[3])
                    flat = ridx * C + cv
                    plsc.store_scatter(out_v[b],
                                       [flat >> 7, flat & 127], acc)

            plsc.parallel_loop(0, CHUNK // LANES, 1)(comb_body)
            fire_out(ci, b)

        fire_pts(jnp.int32(0), 0)
        fire_pts(jnp.int32(1), 1)
        prepare(jnp.int32(0), 0)

        def pair_body(pi, _):
            ci = pi * NBUF
            for b in range(NBUF):
                cur = ci + b
                nxt = cur + 1
                pre = cur + 2

                @pl.when(nxt < CPW)
                def _():
                    prepare(nxt, (b + 1) % NBUF)

                @pl.when(pre < CPW)
                def _():
                    fire_pts(pre, b)

                consume(cur, b)

        lax.fori_loop(0, CPW // NBUF, pair_body, None)
        for b in range(NBUF):
            wait_out(jnp.int32(CPW - NBUF + b), b)

    return grid_kernel(pts_flat, table, coef)


def kernel(points, tar_feature, bbox_min, bbox_max):
    C, D, H, W = tar_feature.shape
    table = tar_feature.reshape(C, D * H * W).T

    scale = jnp.array([W - 1, H - 1, D - 1], jnp.float32) / (bbox_max - bbox_min)
    off = -bbox_min * scale
    coef = jnp.concatenate(
        [jnp.repeat(scale[:, None], LANES, axis=1),
         jnp.repeat(off[:, None], LANES, axis=1),
         jnp.zeros((2, LANES), jnp.float32)], axis=0)

    P = points.shape[0]
    out = _run(points.reshape(-1), table, coef, C=C, D=D, H=H, W=W)
    return out.reshape(P, C)


# speedup vs baseline: 2.4502x; 2.4502x over previous
import functools

import jax
import jax.numpy as jnp
from jax import lax
from jax.experimental import pallas as pl
from jax.experimental.pallas import tpu as pltpu
from jax.experimental.pallas import tpu_sc as plsc

NW = 32
NC = 2
CHUNK = 128
LANES = 16
NBUF = 2


@functools.partial(jax.jit, static_argnames=("C", "D", "H", "W"))
def _run(pts_t, table, coef, *, C, D, H, W):
    P = pts_t.shape[1]
    T = -(-P // CHUNK)
    CPW = -(-(-(-T // NW)) // NBUF) * NBUF
    HW = H * W

    mesh = plsc.VectorSubcoreMesh(core_axis_name="c", subcore_axis_name="s")

    @functools.partial(
        pl.kernel,
        mesh=mesh,
        compiler_params=pltpu.CompilerParams(
            needs_layout_passes=False, use_tc_tiling_on_sc=False),
        out_type=jax.ShapeDtypeStruct((P * C // 128, 128), jnp.float32),
        scratch_types=[
            pltpu.VMEM((8, LANES), jnp.float32),
            [pltpu.VMEM((3, CHUNK), jnp.float32) for _ in range(NBUF)],
            [[pltpu.VMEM((CHUNK,), jnp.int32) for _ in range(8)]
             for _ in range(NBUF)],
            [[pltpu.VMEM((CHUNK,), jnp.float32) for _ in range(8)]
             for _ in range(NBUF)],
            [[pltpu.VMEM((CHUNK, C), jnp.float32) for _ in range(8)]
             for _ in range(NBUF)],
            [pltpu.VMEM((CHUNK * C // 128, 128), jnp.float32)
             for _ in range(NBUF)],
            [pltpu.SemaphoreType.DMA for _ in range(NBUF)],
            [pltpu.SemaphoreType.DMA for _ in range(NBUF)],
            [pltpu.SemaphoreType.DMA for _ in range(NBUF)],
        ],
    )
    def grid_kernel(pts_hbm, table_hbm, coef_hbm, out_hbm,
                    coef_v, pts_v, idx_v, w_v, rows_v, out_v,
                    gsem, psem, osem):
        wid = lax.axis_index("s") * NC + lax.axis_index("c")
        pltpu.sync_copy(coef_hbm, coef_v)
        sxv = coef_v[0, :]
        syv = coef_v[1, :]
        szv = coef_v[2, :]
        oxv = coef_v[3, :]
        oyv = coef_v[4, :]
        ozv = coef_v[5, :]
        iota = lax.iota(jnp.int32, LANES)
        wbase = wid * CPW

        def chunk_base(ci):
            t = jnp.minimum(wbase + ci, T - 1)
            return jnp.minimum(t * CHUNK, P - CHUNK)

        def fire_pts(ci, b):
            base = chunk_base(ci)
            pltpu.async_copy(pts_hbm.at[:, pl.ds(base, CHUNK)], pts_v[b],
                             psem[b])

        def wait_pts(ci, b):
            base = chunk_base(ci)
            pltpu.make_async_copy(pts_hbm.at[:, pl.ds(base, CHUNK)],
                                  pts_v[b], psem[b]).wait()

        def fire_out(ci, b):
            base = chunk_base(ci)
            pltpu.async_copy(out_v[b],
                             out_hbm.at[pl.ds(base * C // 128, CHUNK * C // 128)],
                             osem[b])

        def wait_out(ci, b):
            base = chunk_base(ci)
            pltpu.make_async_copy(
                out_v[b],
                out_hbm.at[pl.ds(base * C // 128, CHUNK * C // 128)],
                osem[b]).wait()

        def prepare(ci, b):
            wait_pts(ci, b)

            def grp_body(g):
                s = g * LANES
                px = pts_v[b][0, pl.ds(s, LANES)]
                py = pts_v[b][1, pl.ds(s, LANES)]
                pz = pts_v[b][2, pl.ds(s, LANES)]
                ix = jnp.clip(px * sxv + oxv, 0.0, float(W - 1))
                iy = jnp.clip(py * syv + oyv, 0.0, float(H - 1))
                iz = jnp.clip(pz * szv + ozv, 0.0, float(D - 1))
                x0 = ix.astype(jnp.int32)
                y0 = iy.astype(jnp.int32)
                z0 = iz.astype(jnp.int32)
                fx = ix - x0.astype(jnp.float32)
                fy = iy - y0.astype(jnp.float32)
                fz = iz - z0.astype(jnp.float32)
                x1 = jnp.minimum(x0 + 1, W - 1)
                y1 = jnp.minimum(y0 + 1, H - 1)
                z1 = jnp.minimum(z0 + 1, D - 1)
                b00 = z0 * HW + y0 * W
                b01 = z0 * HW + y1 * W
                b10 = z1 * HW + y0 * W
                b11 = z1 * HW + y1 * W
                gx = 1.0 - fx
                a = (1.0 - fz) * (1.0 - fy)
                bb = (1.0 - fz) * fy
                c = fz * (1.0 - fy)
                d = fz * fy
                ids = (b00 + x0, b00 + x1, b01 + x0, b01 + x1,
                       b10 + x0, b10 + x1, b11 + x0, b11 + x1)
                ws = (a * gx, a * fx, bb * gx, bb * fx,
                      c * gx, c * fx, d * gx, d * fx)
                for k in range(8):
                    idx_v[b][k][pl.ds(s, LANES)] = ids[k]
                    w_v[b][k][pl.ds(s, LANES)] = ws[k]

            plsc.parallel_loop(0, CHUNK // LANES, 1)(grp_body)
            for k in range(8):
                pltpu.async_copy(table_hbm.at[idx_v[b][k]], rows_v[b][k],
                                 gsem[b])

        def consume(ci, b):
            for k in range(8):
                pltpu.make_async_copy(table_hbm.at[idx_v[b][k]],
                                      rows_v[b][k], gsem[b]).wait()

            @pl.when(ci >= NBUF)
            def _():
                wait_out(ci - NBUF, b)

            def comb_body(g):
                s = g * LANES
                ridx = s + iota
                wv = [w_v[b][k][pl.ds(s, LANES)] for k in range(8)]
                for d in range(C):
                    cv = (iota + d) & (C - 1)
                    ld = [plsc.load_gather(rows_v[b][k], [ridx, cv])
                          for k in range(8)]
                    pr = [wv[2 * j] * ld[2 * j] + wv[2 * j + 1] * ld[2 * j + 1]
                          for j in range(4)]
                    acc = (pr[0] + pr[1]) + (pr[2] + pr[3])
                    flat = ridx * C + cv
                    plsc.store_scatter(out_v[b],
                                       [flat >> 7, flat & 127], acc)

            plsc.parallel_loop(0, CHUNK // LANES, 1)(comb_body)
            fire_out(ci, b)

        fire_pts(jnp.int32(0), 0)
        fire_pts(jnp.int32(1), 1)
        prepare(jnp.int32(0), 0)

        def pair_body(pi, _):
            ci = pi * NBUF
            for b in range(NBUF):
                cur = ci + b
                nxt = cur + 1
                pre = cur + 2

                @pl.when(nxt < CPW)
                def _():
                    prepare(nxt, (b + 1) % NBUF)

                @pl.when(pre < CPW)
                def _():
                    fire_pts(pre, b)

                consume(cur, b)

        lax.fori_loop(0, CPW // NBUF, pair_body, None)
        for b in range(NBUF):
            wait_out(jnp.int32(CPW - NBUF + b), b)

    return grid_kernel(pts_t, table, coef)


def kernel(points, tar_feature, bbox_min, bbox_max):
    C, D, H, W = tar_feature.shape
    table = tar_feature.reshape(C, D * H * W).T

    scale = jnp.array([W - 1, H - 1, D - 1], jnp.float32) / (bbox_max - bbox_min)
    off = -bbox_min * scale
    coef = jnp.concatenate(
        [jnp.repeat(scale[:, None], LANES, axis=1),
         jnp.repeat(off[:, None], LANES, axis=1),
         jnp.zeros((2, LANES), jnp.float32)], axis=0)

    P = points.shape[0]
    out = _run(points.T, table, coef, C=C, D=D, H=H, W=W)
    return out.reshape(P, C)


# speedup vs baseline: 2.4604x; 1.0042x over previous
import functools

import jax
import jax.numpy as jnp
from jax import lax
from jax.experimental import pallas as pl
from jax.experimental.pallas import tpu as pltpu
from jax.experimental.pallas import tpu_sc as plsc

NW = 32
NC = 2
CHUNK = 128
LANES = 16
NBUF = 2


@functools.partial(jax.jit, static_argnames=("C", "D", "H", "W"))
def _run(px_h, py_h, pz_h, table, coef, *, C, D, H, W):
    P = px_h.shape[0]
    T = -(-P // CHUNK)
    CPW = -(-(-(-T // NW)) // NBUF) * NBUF
    HW = H * W

    mesh = plsc.VectorSubcoreMesh(core_axis_name="c", subcore_axis_name="s")

    @functools.partial(
        pl.kernel,
        mesh=mesh,
        compiler_params=pltpu.CompilerParams(
            needs_layout_passes=False, use_tc_tiling_on_sc=False),
        out_type=jax.ShapeDtypeStruct((P * C // 128, 128), jnp.float32),
        scratch_types=[
            pltpu.VMEM((8, LANES), jnp.float32),
            [[pltpu.VMEM((CHUNK,), jnp.float32) for _ in range(3)]
             for _ in range(NBUF)],
            [[pltpu.VMEM((CHUNK,), jnp.int32) for _ in range(8)]
             for _ in range(NBUF)],
            [[pltpu.VMEM((CHUNK,), jnp.float32) for _ in range(8)]
             for _ in range(NBUF)],
            [[pltpu.VMEM((CHUNK, C), jnp.float32) for _ in range(8)]
             for _ in range(NBUF)],
            [pltpu.VMEM((CHUNK * C // 128, 128), jnp.float32)
             for _ in range(NBUF)],
            [pltpu.SemaphoreType.DMA for _ in range(NBUF)],
            [pltpu.SemaphoreType.DMA for _ in range(NBUF)],
            [pltpu.SemaphoreType.DMA for _ in range(NBUF)],
        ],
    )
    def grid_kernel(px_hbm, py_hbm, pz_hbm, table_hbm, coef_hbm, out_hbm,
                    coef_v, pts_v, idx_v, w_v, rows_v, out_v,
                    gsem, psem, osem):
        wid = lax.axis_index("s") * NC + lax.axis_index("c")
        pltpu.sync_copy(coef_hbm, coef_v)
        sxv = coef_v[0, :]
        syv = coef_v[1, :]
        szv = coef_v[2, :]
        oxv = coef_v[3, :]
        oyv = coef_v[4, :]
        ozv = coef_v[5, :]
        iota = lax.iota(jnp.int32, LANES)
        wbase = wid * CPW

        def chunk_base(ci):
            t = jnp.minimum(wbase + ci, T - 1)
            return jnp.minimum(t * CHUNK, P - CHUNK)

        def fire_pts(ci, b):
            base = chunk_base(ci)
            for a, href in enumerate((px_hbm, py_hbm, pz_hbm)):
                pltpu.async_copy(href.at[pl.ds(base, CHUNK)], pts_v[b][a],
                                 psem[b])

        def wait_pts(ci, b):
            base = chunk_base(ci)
            for a, href in enumerate((px_hbm, py_hbm, pz_hbm)):
                pltpu.make_async_copy(href.at[pl.ds(base, CHUNK)],
                                      pts_v[b][a], psem[b]).wait()

        def fire_out(ci, b):
            base = chunk_base(ci)
            pltpu.async_copy(out_v[b],
                             out_hbm.at[pl.ds(base * C // 128, CHUNK * C // 128)],
                             osem[b])

        def wait_out(ci, b):
            base = chunk_base(ci)
            pltpu.make_async_copy(
                out_v[b],
                out_hbm.at[pl.ds(base * C // 128, CHUNK * C // 128)],
                osem[b]).wait()

        def prepare(ci, b):
            wait_pts(ci, b)

            def grp_body(g):
                s = g * LANES
                px = pts_v[b][0][pl.ds(s, LANES)]
                py = pts_v[b][1][pl.ds(s, LANES)]
                pz = pts_v[b][2][pl.ds(s, LANES)]
                ix = jnp.clip(px * sxv + oxv, 0.0, float(W - 1))
                iy = jnp.clip(py * syv + oyv, 0.0, float(H - 1))
                iz = jnp.clip(pz * szv + ozv, 0.0, float(D - 1))
                x0 = ix.astype(jnp.int32)
                y0 = iy.astype(jnp.int32)
                z0 = iz.astype(jnp.int32)
                fx = ix - x0.astype(jnp.float32)
                fy = iy - y0.astype(jnp.float32)
                fz = iz - z0.astype(jnp.float32)
                x1 = jnp.minimum(x0 + 1, W - 1)
                y1 = jnp.minimum(y0 + 1, H - 1)
                z1 = jnp.minimum(z0 + 1, D - 1)
                b00 = z0 * HW + y0 * W
                b01 = z0 * HW + y1 * W
                b10 = z1 * HW + y0 * W
                b11 = z1 * HW + y1 * W
                gx = 1.0 - fx
                a = (1.0 - fz) * (1.0 - fy)
                bb = (1.0 - fz) * fy
                c = fz * (1.0 - fy)
                d = fz * fy
                ids = (b00 + x0, b00 + x1, b01 + x0, b01 + x1,
                       b10 + x0, b10 + x1, b11 + x0, b11 + x1)
                ws = (a * gx, a * fx, bb * gx, bb * fx,
                      c * gx, c * fx, d * gx, d * fx)
                for k in range(8):
                    idx_v[b][k][pl.ds(s, LANES)] = ids[k]
                    w_v[b][k][pl.ds(s, LANES)] = ws[k]

            plsc.parallel_loop(0, CHUNK // LANES, 1)(grp_body)
            for k in range(8):
                pltpu.async_copy(table_hbm.at[idx_v[b][k]], rows_v[b][k],
                                 gsem[b])

        def consume(ci, b):
            for k in range(8):
                pltpu.make_async_copy(table_hbm.at[idx_v[b][k]],
                                      rows_v[b][k], gsem[b]).wait()

            @pl.when(ci >= NBUF)
            def _():
                wait_out(ci - NBUF, b)

            def comb_body(g):
                s = g * LANES
                ridx = s + iota
                wv = [w_v[b][k][pl.ds(s, LANES)] for k in range(8)]
                for d in range(C):
                    cv = (iota + d) & (C - 1)
                    ld = [plsc.load_gather(rows_v[b][k], [ridx, cv])
                          for k in range(8)]
                    pr = [wv[2 * j] * ld[2 * j] + wv[2 * j + 1] * ld[2 * j + 1]
                          for j in range(4)]
                    acc = (pr[0] + pr[1]) + (pr[2] + pr[3])
                    flat = ridx * C + cv
                    plsc.store_scatter(out_v[b],
                                       [flat >> 7, flat & 127], acc)

            plsc.parallel_loop(0, CHUNK // LANES, 1)(comb_body)
            fire_out(ci, b)

        fire_pts(jnp.int32(0), 0)
        fire_pts(jnp.int32(1), 1)
        prepare(jnp.int32(0), 0)

        def pair_body(pi, _):
            ci = pi * NBUF
            for b in range(NBUF):
                cur = ci + b
                nxt = cur + 1
                pre = cur + 2

                @pl.when(nxt < CPW)
                def _():
                    prepare(nxt, (b + 1) % NBUF)

                @pl.when(pre < CPW)
                def _():
                    fire_pts(pre, b)

                consume(cur, b)

        lax.fori_loop(0, CPW // NBUF, pair_body, None)
        for b in range(NBUF):
            wait_out(jnp.int32(CPW - NBUF + b), b)

    return grid_kernel(px_h, py_h, pz_h, table, coef)


def kernel(points, tar_feature, bbox_min, bbox_max):
    C, D, H, W = tar_feature.shape
    table = tar_feature.reshape(C, D * H * W).T

    scale = jnp.array([W - 1, H - 1, D - 1], jnp.float32) / (bbox_max - bbox_min)
    off = -bbox_min * scale
    coef = jnp.concatenate(
        [jnp.repeat(scale[:, None], LANES, axis=1),
         jnp.repeat(off[:, None], LANES, axis=1),
         jnp.zeros((2, LANES), jnp.float32)], axis=0)

    P = points.shape[0]
    out = _run(points[:, 0], points[:, 1], points[:, 2], table, coef,
               C=C, D=D, H=H, W=W)
    return out.reshape(P, C)
